# baseline (speedup 1.0000x reference)
import functools

import jax
import jax.numpy as jnp
from jax import lax
from jax.experimental import pallas as pl
from jax.experimental.pallas import tpu as pltpu
from jax.experimental.pallas import tpu_sc as plsc

BATCH = 16384
LATENT_DIM = 16
ROWS_PER_SUP = 128 // LATENT_DIM
NUM_CORES = 2
NUM_SUBCORES = 16
NUM_WORKERS = NUM_CORES * NUM_SUBCORES
ROWS_PER_WORKER = BATCH // NUM_WORKERS
IDX_CHUNK = 128
NUM_CHUNKS = ROWS_PER_WORKER // IDX_CHUNK
GROUPS_PER_CHUNK = IDX_CHUNK // LATENT_DIM


def _gmf_body(usup_hbm, isup_hbm, uoff_hbm, ioff_hbm, emb_u_hbm, emb_i_hbm,
              wrot_hbm, b_hbm, out_hbm,
              usup_v, isup_v, uoff_v, ioff_v, u_buf, i_buf, wrot_v, b_v,
              out_v, sem0, sem1):
    wid = lax.axis_index("c") * NUM_SUBCORES + lax.axis_index("s")
    sems = (sem0, sem1)

    pltpu.sync_copy(usup_hbm.at[wid], usup_v)
    pltpu.sync_copy(isup_hbm.at[wid], isup_v)
    pltpu.sync_copy(uoff_hbm.at[wid], uoff_v)
    pltpu.sync_copy(ioff_hbm.at[wid], ioff_v)
    pltpu.sync_copy(wrot_hbm, wrot_v)
    pltpu.sync_copy(b_hbm, b_v)

    def fire(j):
        slot = j & 1
        return (
            pltpu.async_copy(emb_u_hbm.at[usup_v.at[j]], u_buf.at[slot], sems[slot]),
            pltpu.async_copy(emb_i_hbm.at[isup_v.at[j]], i_buf.at[slot], sems[slot]),
        )

    iota16 = lax.iota(jnp.int32, 16)
    colb = [(iota16 + d) & 15 for d in range(LATENT_DIM)]
    w_rots = [wrot_v[d] for d in range(LATENT_DIM)]
    b_reg = b_v[...]

    pending = fire(0)
    for j in range(NUM_CHUNKS):
        slot = j & 1
        for d in pending:
            d.wait()
        if j + 1 < NUM_CHUNKS:
            pending = fire(j + 1)

        ub, ib = u_buf.at[slot], i_buf.at[slot]

        def group(g, carry):
            row_ids = g * 16 + iota16
            u_off = uoff_v[pl.ds(j * IDX_CHUNK + g * 16, 16)]
            i_off = ioff_v[pl.ds(j * IDX_CHUNK + g * 16, 16)]
            acc = jnp.zeros((16,), jnp.float32)
            for d in range(LATENT_DIM):
                uc = plsc.load_gather(ub, [row_ids, u_off + colb[d]])
                ic = plsc.load_gather(ib, [row_ids, i_off + colb[d]])
                acc = acc + uc * ic * w_rots[d]
            logits = acc + b_reg
            rating = 1.0 / (1.0 + jnp.exp(-logits))
            out_v[pl.ds(j * IDX_CHUNK + g * 16, 16)] = rating
            return carry

        lax.fori_loop(0, GROUPS_PER_CHUNK, group, 0)

    pltpu.sync_copy(out_v, out_hbm.at[pl.ds(wid * ROWS_PER_WORKER, ROWS_PER_WORKER)])


@jax.jit
def _gmf(usup, isup, uoff, ioff, emb_u2, emb_i2, wrot, b16):
    mesh = plsc.VectorSubcoreMesh(core_axis_name="c", subcore_axis_name="s")
    f = functools.partial(
        pl.kernel,
        mesh=mesh,
        out_type=jax.ShapeDtypeStruct((BATCH,), jnp.float32),
        compiler_params=pltpu.CompilerParams(needs_layout_passes=False),
        scratch_types=[
            pltpu.VMEM((NUM_CHUNKS, IDX_CHUNK), jnp.int32),
            pltpu.VMEM((NUM_CHUNKS, IDX_CHUNK), jnp.int32),
            pltpu.VMEM((ROWS_PER_WORKER,), jnp.int32),
            pltpu.VMEM((ROWS_PER_WORKER,), jnp.int32),
            pltpu.VMEM((2, IDX_CHUNK, 128), jnp.float32),
            pltpu.VMEM((2, IDX_CHUNK, 128), jnp.float32),
            pltpu.VMEM((LATENT_DIM, LATENT_DIM), jnp.float32),
            pltpu.VMEM((LATENT_DIM,), jnp.float32),
            pltpu.VMEM((ROWS_PER_WORKER,), jnp.float32),
            pltpu.SemaphoreType.DMA,
            pltpu.SemaphoreType.DMA,
        ],
    )(_gmf_body)
    return f(usup, isup, uoff, ioff, emb_u2, emb_i2, wrot, b16)


def kernel(user_indices, item_indices, domain_idc, embedding_user,
           embedding_item, affine_W, affine_b):
    del domain_idc
    usup = (user_indices // ROWS_PER_SUP).reshape(NUM_WORKERS, NUM_CHUNKS, IDX_CHUNK)
    isup = (item_indices // ROWS_PER_SUP).reshape(NUM_WORKERS, NUM_CHUNKS, IDX_CHUNK)
    uoff = ((user_indices % ROWS_PER_SUP) * LATENT_DIM).reshape(NUM_WORKERS, ROWS_PER_WORKER)
    ioff = ((item_indices % ROWS_PER_SUP) * LATENT_DIM).reshape(NUM_WORKERS, ROWS_PER_WORKER)
    emb_u2 = embedding_user.reshape(-1, 128)
    emb_i2 = embedding_item.reshape(-1, 128)
    w16 = affine_W.reshape(LATENT_DIM)
    wrot = jnp.stack([jnp.roll(w16, -d) for d in range(LATENT_DIM)])
    b16 = jnp.broadcast_to(affine_b, (LATENT_DIM,))
    out = _gmf(usup, isup, uoff, ioff, emb_u2, emb_i2, wrot, b16)
    return out.reshape(BATCH, 1)
